# baseline (speedup 1.0000x reference)
import functools

import jax
import jax.numpy as jnp
from jax import lax
from jax.experimental import pallas as pl
from jax.experimental.pallas import tpu as pltpu
from jax.experimental.pallas import tpu_sc as plsc

_B, _C, _N = 8, 3, 2048
_R = 256
_NB = _N // _R
_INF = float("inf")

_NW = 32
_ROWS_PER_W = (_B * _N) // _NW
_BLK = 8
_NBLK = _ROWS_PER_W // _BLK


def _pd_body(x_ref, q_ref, pd_ref):
    xb = x_ref[0]
    q = q_ref[0]
    inner = -2.0 * jnp.dot(q, xb, preferred_element_type=jnp.float32)
    xxj = xb[0:1, :] * xb[0:1, :] + xb[1:2, :] * xb[1:2, :] + xb[2:3, :] * xb[2:3, :]
    xxi = q[:, 0:1] * q[:, 0:1] + q[:, 1:2] * q[:, 1:2] + q[:, 2:3] * q[:, 2:3]
    pd_ref[...] = 0.0 - (((0.0 - xxj) - inner) - xxi)


def _pd_call(x0, ptab):
    return pl.pallas_call(
        _pd_body,
        grid=(_B, _NB),
        in_specs=[
            pl.BlockSpec((1, _C, _N), lambda b, r: (b, 0, 0)),
            pl.BlockSpec((1, _R, 3), lambda b, r: (b, r, 0)),
        ],
        out_specs=pl.BlockSpec((_R, _N), lambda b, r: (b * _NB + r, 0)),
        out_shape=jax.ShapeDtypeStruct((_B * _N, _N), jnp.float32),
    )(x0, ptab)


def _merge16_asc(a, b):
    rb = lax.rev(b, (0,))
    lo = jnp.minimum(a, rb)
    hi = jnp.maximum(a, rb)
    return jnp.sort(lo), jnp.sort(hi)


def _merge32_low(a0, a1, b0, b1):
    l0 = jnp.minimum(a0, lax.rev(b1, (0,)))
    l1 = jnp.minimum(a1, lax.rev(b0, (0,)))
    p = jnp.minimum(l0, l1)
    q = jnp.maximum(l0, l1)
    return jnp.sort(p), jnp.sort(q)


def _thresh32(cmins):
    s = [jnp.sort(c) for c in cmins]
    m = [_merge16_asc(s[2 * i], s[2 * i + 1]) for i in range(4)]
    u0 = _merge32_low(*m[0], *m[1])
    u1 = _merge32_low(*m[2], *m[3])
    l0 = jnp.minimum(u0[0], lax.rev(u1[1], (0,)))
    l1 = jnp.minimum(u0[1], lax.rev(u1[0], (0,)))
    return jnp.max(jnp.maximum(l0, l1))


def _sc_body(pd_hbm, x_hbm, out_hbm, fbuf, rbuf, sbuf, ibuf, obuf):
    wid = lax.axis_index("s") * 2 + lax.axis_index("c")
    b = wid // 4
    grow = b * _N + (wid % 4) * _ROWS_PER_W

    pltpu.sync_copy(x_hbm.at[b], fbuf)
    iota = lax.iota(jnp.int32, 16)
    inf16 = jnp.full((16,), _INF, jnp.float32)

    def blk_body(blk, carry):
        pltpu.sync_copy(
            pd_hbm.at[pl.ds((grow + blk * _BLK) * _N, _BLK * _N)], rbuf)

        def row_body(r, carry2):
            rbase = r * _N
            cmins = []
            for g in range(8):
                base = rbase + g * 256 + iota * 16

                def jstep(j, acc):
                    return jnp.minimum(
                        acc, plsc.load_gather(rbuf, [base + j]))

                cmins.append(lax.fori_loop(0, 16, jstep, inf16))
            t = _thresh32(cmins)

            def cstep(c, off):
                v = rbuf[pl.ds(rbase + c * 16, 16)]
                m = v <= t
                plsc.store_compressed(sbuf.at[pl.ds(off, 16)], v, mask=m)
                plsc.store_compressed(ibuf.at[pl.ds(off, 16)], c * 16 + iota, mask=m)
                return off + jnp.sum(m.astype(jnp.int32))

            cnt = lax.fori_loop(0, _N // 16, cstep, jnp.int32(0))
            sbuf[pl.ds(cnt, 16)] = inf16

            def mstep(tr, cur):
                c0, c1, i0, i1 = cur
                v = sbuf[pl.ds(tr * 16, 16)]
                vi = ibuf[pl.ds(tr * 16, 16)]
                v, vi = plsc.sort_key_val(v, vi)
                rv = lax.rev(v, (0,))
                rvi = lax.rev(vi, (0,))
                m = c1 <= rv
                l1 = jnp.where(m, c1, rv)
                li1 = jnp.where(m, i1, rvi)
                m2 = c0 <= l1
                p = jnp.where(m2, c0, l1)
                pi = jnp.where(m2, i0, li1)
                q = jnp.where(m2, l1, c0)
                qi = jnp.where(m2, li1, i0)
                p, pi = plsc.sort_key_val(p, pi)
                q, qi = plsc.sort_key_val(q, qi)
                return p, q, pi, qi

            ntrip = (cnt + 15) // 16
            zero16 = jnp.zeros((16,), jnp.int32)
            _, _, i0, i1 = lax.fori_loop(
                0, ntrip, mstep, (inf16, inf16, zero16, zero16))

            g0x = plsc.load_gather(fbuf, [i0])
            g0y = plsc.load_gather(fbuf, [i0 + 2048])
            g0z = plsc.load_gather(fbuf, [i0 + 4096])
            g1x = plsc.load_gather(fbuf, [i1])
            g1y = plsc.load_gather(fbuf, [i1 + 2048])
            g1z = plsc.load_gather(fbuf, [i1 + 4096])
            si = i0 * 3
            s16x = plsc.load_gather(fbuf, [si])
            s16y = plsc.load_gather(fbuf, [si + 1])
            s16z = plsc.load_gather(fbuf, [si + 2])

            lo8 = iota < 8
            zf = jnp.float32(0.0)
            s0x, s0y, s0z = jnp.sum(g0x), jnp.sum(g0y), jnp.sum(g0z)
            vals = (
                jnp.sum(jnp.where(lo8, g0x, zf)) * 0.125,
                jnp.sum(jnp.where(lo8, g0y, zf)) * 0.125,
                jnp.sum(jnp.where(lo8, g0z, zf)) * 0.125,
                jnp.sum(s16x) * 0.0625,
                jnp.sum(s16y) * 0.0625,
                jnp.sum(s16z) * 0.0625,
                (s0x + jnp.sum(g1x)) * 0.03125,
                (s0y + jnp.sum(g1y)) * 0.03125,
                (s0z + jnp.sum(g1z)) * 0.03125,
            )
            ovec = jnp.zeros((16,), jnp.float32)
            for c, v in enumerate(vals):
                ovec = jnp.where(iota == c, v, ovec)
            obuf[pl.ds((blk * _BLK + r) * 16, 16)] = ovec
            return carry2

        return lax.fori_loop(0, _BLK, row_body, carry)

    lax.fori_loop(0, _NBLK, blk_body, jnp.int32(0))
    pltpu.sync_copy(obuf, out_hbm.at[pl.ds(grow * 16, _ROWS_PER_W * 16)])


@functools.partial(
    pl.kernel,
    out_type=jax.ShapeDtypeStruct((_B * _N * 16,), jnp.float32),
    mesh=plsc.VectorSubcoreMesh(core_axis_name="c", subcore_axis_name="s"),
    compiler_params=pltpu.CompilerParams(needs_layout_passes=False),
    scratch_types=[
        pltpu.VMEM((_C * _N,), jnp.float32),
        pltpu.VMEM((_BLK * _N,), jnp.float32),
        pltpu.VMEM((_N + 16,), jnp.float32),
        pltpu.VMEM((_N + 16,), jnp.int32),
        pltpu.VMEM((_ROWS_PER_W * 16,), jnp.float32),
    ],
)
def _sc_knn(pd_hbm, x_hbm, out_hbm, fbuf, rbuf, sbuf, ibuf, obuf):
    _sc_body(pd_hbm, x_hbm, out_hbm, fbuf, rbuf, sbuf, ibuf, obuf)


def kernel(x):
    batch_size = x.shape[0]
    num_points = x.shape[3]
    x0 = jnp.reshape(x, (batch_size, -1, num_points))
    ptab = jnp.swapaxes(x0, 1, 2)

    sdist = _pd_call(x0, ptab)
    feats = _sc_knn(jnp.reshape(sdist, (_B * _N * _N,)),
                    jnp.reshape(x0, (batch_size, _C * _N)))
    feats = feats.reshape(batch_size, num_points, 16)
    f8, f16, f32 = feats[..., 0:3], feats[..., 3:6], feats[..., 6:9]

    concat_x = jnp.swapaxes(jnp.expand_dims(x0, 3), 2, 1)
    for feat in (f8, f16, f32):
        feature = feat.reshape(batch_size, num_points, 1, 1, 3)
        num_dims = concat_x.shape[3]
        concat_x = jnp.reshape(concat_x, (batch_size, num_points, 1, num_dims, 3))
        concat_x = jnp.concatenate((feature, concat_x), axis=3)
        concat_x = jnp.transpose(concat_x, (0, 4, 1, 3, 2))
    concat_x = jnp.transpose(concat_x, (0, 3, 1, 2, 4))
    return concat_x
